# baseline (speedup 1.0000x reference)
import functools

import jax
import jax.numpy as jnp
from jax import lax
from jax.experimental import pallas as pl
from jax.experimental.pallas import tpu as pltpu
from jax.experimental.pallas import tpu_sc as plsc

NUM_CORES = 2
NUM_SUBCORES = 16
NW = NUM_CORES * NUM_SUBCORES

CHUNK = 800
NBUF = 2
PAD_H = 128


def _gather_kernel(idx_hbm, table_hbm, out_hbm, idx_v, *bufs_and_sems):
    rows = bufs_and_sems[:NBUF]
    sg = bufs_and_sems[NBUF:2 * NBUF]
    so = bufs_and_sems[2 * NBUF:3 * NBUF]

    n_chunks_total = idx_hbm.shape[0]
    n_chunks = n_chunks_total // NW
    wid = lax.axis_index("s") * NUM_CORES + lax.axis_index("c")
    base = wid * n_chunks

    pltpu.sync_copy(idx_hbm.at[pl.ds(base, n_chunks)], idx_v)

    def fire_gather(i, b):
        pltpu.async_copy(table_hbm.at[idx_v.at[i]], rows[b], sg[b])

    def wait_gather(b):
        pltpu.make_async_copy(table_hbm.at[idx_v.at[0]], rows[b], sg[b]).wait()

    def fire_out(i, b):
        pltpu.async_copy(rows[b], out_hbm.at[base + i], so[b])

    def wait_out(b):
        pltpu.make_async_copy(rows[b], out_hbm.at[base], so[b]).wait()

    for b in range(NBUF):
        fire_gather(b, b)

    @pl.loop(0, n_chunks // NBUF)
    def _group(g):
        i0 = g * NBUF
        for b in range(NBUF):
            i = i0 + b
            wait_gather(b)
            fire_out(i, b)

            @pl.when(i + NBUF < n_chunks)
            def _():
                wait_out(b)
                fire_gather(i + NBUF, b)

    for b in range(NBUF):
        wait_out(b)


@functools.partial(jax.jit, static_argnums=(2,))
def _gather(idx2d, weight, total):
    mesh = plsc.VectorSubcoreMesh(
        core_axis_name="c",
        subcore_axis_name="s",
        num_cores=NUM_CORES,
        num_subcores=NUM_SUBCORES,
    )
    hidden = weight.shape[1]
    n_chunks_total = idx2d.shape[0]
    n_chunks = n_chunks_total // NW
    scratch = [pltpu.VMEM((n_chunks, CHUNK), jnp.int32)]
    scratch += [pltpu.VMEM((CHUNK, hidden), jnp.float32) for _ in range(NBUF)]
    scratch += [pltpu.SemaphoreType.DMA for _ in range(2 * NBUF)]
    return pl.kernel(
        _gather_kernel,
        out_type=jax.ShapeDtypeStruct(
            (n_chunks_total, CHUNK, hidden), weight.dtype
        ),
        mesh=mesh,
        compiler_params=pltpu.CompilerParams(use_tc_tiling_on_sc=False),
        scratch_types=scratch,
    )(idx2d, weight)


def kernel(inputs, weight):
    batch, hist = inputs.shape
    total = batch * hist
    idx2d = inputs.reshape(total // CHUNK, CHUNK).astype(jnp.int32)
    out = _gather(idx2d, weight, total)
    return out.reshape(batch, hist, weight.shape[1])


# speedup vs baseline: 1.2234x; 1.2234x over previous
import functools

import jax
import jax.numpy as jnp
from jax import lax
from jax.experimental import pallas as pl
from jax.experimental.pallas import tpu as pltpu
from jax.experimental.pallas import tpu_sc as plsc

NUM_CORES = 2
NUM_SUBCORES = 16
NW = NUM_CORES * NUM_SUBCORES

CHUNK = 400
NBUF = 2
PAD_H = 128


def _gather_kernel(idx_hbm, table_hbm, out_hbm, idx_v, *bufs_and_sems):
    rows = bufs_and_sems[:NBUF]
    sg = bufs_and_sems[NBUF:2 * NBUF]
    so = bufs_and_sems[2 * NBUF:3 * NBUF]

    n_chunks_total = idx_hbm.shape[0]
    n_chunks = n_chunks_total // NW
    wid = lax.axis_index("s") * NUM_CORES + lax.axis_index("c")
    base = wid * n_chunks

    pltpu.sync_copy(idx_hbm.at[pl.ds(base, n_chunks)], idx_v)

    def fire_gather(i, b):
        pltpu.async_copy(table_hbm.at[idx_v.at[i]], rows[b], sg[b])

    def wait_gather(b):
        pltpu.make_async_copy(table_hbm.at[idx_v.at[0]], rows[b], sg[b]).wait()

    def fire_out(i, b):
        pltpu.async_copy(rows[b], out_hbm.at[base + i], so[b])

    def wait_out(b):
        pltpu.make_async_copy(rows[b], out_hbm.at[base], so[b]).wait()

    for b in range(NBUF):
        fire_gather(b, b)

    @pl.loop(0, n_chunks // NBUF)
    def _group(g):
        i0 = g * NBUF
        for b in range(NBUF):
            i = i0 + b
            wait_gather(b)
            fire_out(i, b)

            @pl.when(i + NBUF < n_chunks)
            def _():
                wait_out(b)
                fire_gather(i + NBUF, b)

    for b in range(NBUF):
        wait_out(b)


@functools.partial(jax.jit, static_argnums=(2,))
def _gather(idx2d, weight, total):
    mesh = plsc.VectorSubcoreMesh(
        core_axis_name="c",
        subcore_axis_name="s",
        num_cores=NUM_CORES,
        num_subcores=NUM_SUBCORES,
    )
    hidden = weight.shape[1]
    weight = jnp.pad(weight, ((0, 0), (0, PAD_H - hidden)))
    n_chunks_total = idx2d.shape[0]
    n_chunks = n_chunks_total // NW
    scratch = [pltpu.VMEM((n_chunks, CHUNK), jnp.int32)]
    scratch += [pltpu.VMEM((CHUNK, PAD_H), jnp.float32) for _ in range(NBUF)]
    scratch += [pltpu.SemaphoreType.DMA for _ in range(2 * NBUF)]
    return pl.kernel(
        _gather_kernel,
        out_type=jax.ShapeDtypeStruct(
            (n_chunks_total, CHUNK, PAD_H), weight.dtype
        ),
        mesh=mesh,
        compiler_params=pltpu.CompilerParams(use_tc_tiling_on_sc=False),
        scratch_types=scratch,
    )(idx2d, weight)


def kernel(inputs, weight):
    batch, hist = inputs.shape
    total = batch * hist
    idx2d = inputs.reshape(total // CHUNK, CHUNK).astype(jnp.int32)
    out = _gather(idx2d, weight, total)
    return out.reshape(batch, hist, PAD_H)[:, :, : weight.shape[1]]


# speedup vs baseline: 1.3297x; 1.0869x over previous
import functools

import jax
import jax.numpy as jnp
from jax import lax
from jax.experimental import pallas as pl
from jax.experimental.pallas import tpu as pltpu
from jax.experimental.pallas import tpu_sc as plsc

NUM_CORES = 2
NUM_SUBCORES = 16
NW = NUM_CORES * NUM_SUBCORES

CHUNK = 800
NBUF = 2
PAD_H = 128


def _gather_kernel(idx_hbm, table_hbm, out_hbm, idx_v, *bufs_and_sems):
    rows = bufs_and_sems[:NBUF]
    sg = bufs_and_sems[NBUF:2 * NBUF]
    so = bufs_and_sems[2 * NBUF:3 * NBUF]

    n_chunks_total = idx_hbm.shape[0]
    n_chunks = n_chunks_total // NW
    wid = lax.axis_index("s") * NUM_CORES + lax.axis_index("c")
    base = wid * n_chunks

    pltpu.sync_copy(idx_hbm.at[pl.ds(base, n_chunks)], idx_v)

    def fire_gather(i, b):
        pltpu.async_copy(table_hbm.at[idx_v.at[i]], rows[b], sg[b])

    def wait_gather(b):
        pltpu.make_async_copy(table_hbm.at[idx_v.at[0]], rows[b], sg[b]).wait()

    def fire_out(i, b):
        pltpu.async_copy(
            rows[b], out_hbm.at[base + i, :, pl.ds(0, 64)], so[b]
        )

    def wait_out(b):
        pltpu.make_async_copy(
            rows[b], out_hbm.at[base, :, pl.ds(0, 64)], so[b]
        ).wait()

    for b in range(NBUF):
        fire_gather(b, b)

    @pl.loop(0, n_chunks // NBUF)
    def _group(g):
        i0 = g * NBUF
        for b in range(NBUF):
            i = i0 + b
            wait_gather(b)
            fire_out(i, b)

            @pl.when(i + NBUF < n_chunks)
            def _():
                wait_out(b)
                fire_gather(i + NBUF, b)

    for b in range(NBUF):
        wait_out(b)


@functools.partial(jax.jit, static_argnums=(2,))
def _gather(idx2d, weight, total):
    mesh = plsc.VectorSubcoreMesh(
        core_axis_name="c",
        subcore_axis_name="s",
        num_cores=NUM_CORES,
        num_subcores=NUM_SUBCORES,
    )
    hidden = weight.shape[1]
    n_chunks_total = idx2d.shape[0]
    n_chunks = n_chunks_total // NW
    scratch = [pltpu.VMEM((n_chunks, CHUNK), jnp.int32)]
    scratch += [pltpu.VMEM((CHUNK, hidden), jnp.float32) for _ in range(NBUF)]
    scratch += [pltpu.SemaphoreType.DMA for _ in range(2 * NBUF)]
    return pl.kernel(
        _gather_kernel,
        out_type=jax.ShapeDtypeStruct(
            (n_chunks_total, CHUNK, PAD_H), weight.dtype
        ),
        mesh=mesh,
        compiler_params=pltpu.CompilerParams(use_tc_tiling_on_sc=False),
        scratch_types=scratch,
    )(idx2d, weight)


def kernel(inputs, weight):
    batch, hist = inputs.shape
    total = batch * hist
    idx2d = inputs.reshape(total // CHUNK, CHUNK).astype(jnp.int32)
    out = _gather(idx2d, weight, total)
    return out.reshape(batch, hist, PAD_H)[:, :, : weight.shape[1]]
